# baseline (device time: 50186 ns/iter reference)
import jax
import jax.numpy as jnp
from jax import lax
from jax.experimental import pallas as pl
from jax.experimental.pallas import tpu as pltpu

N_DEV = 8
B, SQ = 2, 128
BLK = 64
HQ_LOC, DH = 4, 64
DMODEL = 512
DCHUNK = HQ_LOC * DH


def kernel(x, Wq, K_ext, V_ext, Wo):
    def body(x_ref, wq_ref, k_ref, v_ref, wo_ref, out_ref,
             ctx_all, send_sems, recv_sems):
        my = lax.axis_index("i")
        left = lax.rem(my - 1 + N_DEV, N_DEV)
        right = lax.rem(my + 1, N_DEV)

        barrier_sem = pltpu.get_barrier_semaphore()
        for nbr in (left, right):
            pl.semaphore_signal(
                barrier_sem, inc=1,
                device_id=(nbr,), device_id_type=pl.DeviceIdType.MESH,
            )
        pl.semaphore_wait(barrier_sem, 2)

        x2d = x_ref[...].reshape(B * SQ, DMODEL)
        wq_slice = wq_ref[:, pl.ds(my * DCHUNK, DCHUNK)]
        q = jnp.dot(x2d, wq_slice, preferred_element_type=jnp.float32)

        row_chunks = []
        for b in range(B):
            for t in range(SQ // BLK):
                r0 = b * SQ + t * BLK
                head_chunks = []
                for h in range(HQ_LOC):
                    qb = q[r0:r0 + BLK, h * DH:(h + 1) * DH]
                    kb = k_ref[b, t * BLK:(t + 1) * BLK, h, :]
                    vb = v_ref[b, t * BLK:(t + 1) * BLK, h, :]
                    s = lax.dot_general(
                        qb, kb, (((1,), (1,)), ((), ())),
                        preferred_element_type=jnp.float32,
                    ) * 0.125
                    s = s - jnp.max(s, axis=-1, keepdims=True)
                    w = jnp.exp(s)
                    w = w / jnp.sum(w, axis=-1, keepdims=True)
                    head_chunks.append(
                        jnp.dot(w, vb, preferred_element_type=jnp.float32)
                    )
                row_chunks.append(jnp.concatenate(head_chunks, axis=1))
        ctx_local = jnp.concatenate(row_chunks, axis=0)
        ctx_all[my] = ctx_local

        for hop in range(N_DEV - 1):
            src_idx = lax.rem(my - hop + N_DEV, N_DEV)
            rdma = pltpu.make_async_remote_copy(
                src_ref=ctx_all.at[src_idx],
                dst_ref=ctx_all.at[src_idx],
                send_sem=send_sems.at[hop],
                recv_sem=recv_sems.at[hop],
                device_id=(right,),
                device_id_type=pl.DeviceIdType.MESH,
            )
            rdma.start()
            rdma.wait()

        acc = jnp.zeros((B * SQ, DMODEL), jnp.float32)
        for i in range(N_DEV):
            acc = acc + jnp.dot(
                ctx_all[i],
                wo_ref[i * DCHUNK:(i + 1) * DCHUNK, :],
                preferred_element_type=jnp.float32,
            )
        out_ref[...] = acc.reshape(B, SQ, DMODEL)

    return pl.pallas_call(
        body,
        out_shape=jax.ShapeDtypeStruct((B, SQ, DMODEL), jnp.float32),
        in_specs=[pl.BlockSpec(memory_space=pltpu.VMEM)] * 5,
        out_specs=pl.BlockSpec(memory_space=pltpu.VMEM),
        scratch_shapes=[
            pltpu.VMEM((N_DEV, B * SQ, DCHUNK), jnp.float32),
            pltpu.SemaphoreType.DMA((N_DEV - 1,)),
            pltpu.SemaphoreType.DMA((N_DEV - 1,)),
        ],
        compiler_params=pltpu.CompilerParams(collective_id=0),
    )(x, Wq, K_ext, V_ext, Wo)


# device time: 29008 ns/iter; 1.7301x vs baseline; 1.7301x over previous
import jax
import jax.numpy as jnp
from jax import lax
from jax.experimental import pallas as pl
from jax.experimental.pallas import tpu as pltpu

N_DEV = 8
B, SQ = 2, 128
BLK = 64
HQ_LOC, DH = 4, 64
DMODEL = 512
DCHUNK = HQ_LOC * DH
ROWS = (B * SQ) // N_DEV


def kernel(x, Wq, K_ext, V_ext, Wo):
    def body(x_ref, wq_ref, k_ref, v_ref, wo_ref, out_ref,
             partial_buf, rs_recv, out2d,
             rs_send_sems, rs_recv_sems, ag_send_sems, ag_recv_sems):
        my = lax.axis_index("i")

        barrier_sem = pltpu.get_barrier_semaphore()
        for d in range(1, N_DEV):
            tgt = lax.rem(my + d, N_DEV)
            pl.semaphore_signal(
                barrier_sem, inc=1,
                device_id=(tgt,), device_id_type=pl.DeviceIdType.MESH,
            )
        pl.semaphore_wait(barrier_sem, N_DEV - 1)

        x2d = x_ref[...].reshape(B * SQ, DMODEL)
        wq_slice = wq_ref[:, pl.ds(my * DCHUNK, DCHUNK)]
        q = jnp.dot(x2d, wq_slice, preferred_element_type=jnp.float32)

        row_chunks = []
        for b in range(B):
            for t in range(SQ // BLK):
                r0 = b * SQ + t * BLK
                head_chunks = []
                for h in range(HQ_LOC):
                    qb = q[r0:r0 + BLK, h * DH:(h + 1) * DH]
                    kb = k_ref[b, t * BLK:(t + 1) * BLK, h, :]
                    vb = v_ref[b, t * BLK:(t + 1) * BLK, h, :]
                    s = lax.dot_general(
                        qb, kb, (((1,), (1,)), ((), ())),
                        preferred_element_type=jnp.float32,
                    ) * 0.125
                    s = s - jnp.max(s, axis=-1, keepdims=True)
                    w = jnp.exp(s)
                    w = w / jnp.sum(w, axis=-1, keepdims=True)
                    head_chunks.append(
                        jnp.dot(w, vb, preferred_element_type=jnp.float32)
                    )
                row_chunks.append(jnp.concatenate(head_chunks, axis=1))
        ctx_local = jnp.concatenate(row_chunks, axis=0)

        wo_slice = wo_ref[pl.ds(my * DCHUNK, DCHUNK), :]
        partial_buf[...] = jnp.dot(
            ctx_local, wo_slice, preferred_element_type=jnp.float32
        )

        rs_sends = []
        for d in range(1, N_DEV):
            tgt = lax.rem(my + d, N_DEV)
            rdma = pltpu.make_async_remote_copy(
                src_ref=partial_buf.at[pl.ds(tgt * ROWS, ROWS), :],
                dst_ref=rs_recv.at[d],
                send_sem=rs_send_sems.at[d - 1],
                recv_sem=rs_recv_sems.at[d - 1],
                device_id=(tgt,),
                device_id_type=pl.DeviceIdType.MESH,
            )
            rdma.start()
            rs_sends.append(rdma)

        acc = partial_buf[pl.ds(my * ROWS, ROWS), :]
        for d in range(1, N_DEV):
            recv = pltpu.make_async_remote_copy(
                src_ref=rs_recv.at[d],
                dst_ref=rs_recv.at[d],
                send_sem=rs_send_sems.at[d - 1],
                recv_sem=rs_recv_sems.at[d - 1],
                device_id=(my,),
                device_id_type=pl.DeviceIdType.MESH,
            )
            recv.wait_recv()
            acc = acc + rs_recv[d]
        out2d[pl.ds(my * ROWS, ROWS), :] = acc
        for rdma in rs_sends:
            rdma.wait_send()

        ag_sends = []
        for d in range(1, N_DEV):
            tgt = lax.rem(my + d, N_DEV)
            rdma = pltpu.make_async_remote_copy(
                src_ref=out2d.at[pl.ds(my * ROWS, ROWS), :],
                dst_ref=out2d.at[pl.ds(my * ROWS, ROWS), :],
                send_sem=ag_send_sems.at[d - 1],
                recv_sem=ag_recv_sems.at[d - 1],
                device_id=(tgt,),
                device_id_type=pl.DeviceIdType.MESH,
            )
            rdma.start()
            ag_sends.append(rdma)
        for d in range(1, N_DEV):
            snd = lax.rem(my - d + N_DEV, N_DEV)
            recv = pltpu.make_async_remote_copy(
                src_ref=out2d.at[pl.ds(snd * ROWS, ROWS), :],
                dst_ref=out2d.at[pl.ds(snd * ROWS, ROWS), :],
                send_sem=ag_send_sems.at[d - 1],
                recv_sem=ag_recv_sems.at[d - 1],
                device_id=(snd,),
                device_id_type=pl.DeviceIdType.MESH,
            )
            recv.wait_recv()
        for rdma in ag_sends:
            rdma.wait_send()

        out_ref[...] = out2d[...].reshape(B, SQ, DMODEL)

    return pl.pallas_call(
        body,
        out_shape=jax.ShapeDtypeStruct((B, SQ, DMODEL), jnp.float32),
        in_specs=[pl.BlockSpec(memory_space=pltpu.VMEM)] * 5,
        out_specs=pl.BlockSpec(memory_space=pltpu.VMEM),
        scratch_shapes=[
            pltpu.VMEM((B * SQ, DMODEL), jnp.float32),
            pltpu.VMEM((N_DEV, ROWS, DMODEL), jnp.float32),
            pltpu.VMEM((B * SQ, DMODEL), jnp.float32),
            pltpu.SemaphoreType.DMA((N_DEV - 1,)),
            pltpu.SemaphoreType.DMA((N_DEV,)),
            pltpu.SemaphoreType.DMA((N_DEV - 1,)),
            pltpu.SemaphoreType.DMA((N_DEV,)),
        ],
        compiler_params=pltpu.CompilerParams(collective_id=0),
    )(x, Wq, K_ext, V_ext, Wo)


# device time: 22365 ns/iter; 2.2440x vs baseline; 1.2970x over previous
import jax
import jax.numpy as jnp
from jax import lax
from jax.experimental import pallas as pl
from jax.experimental.pallas import tpu as pltpu

N_DEV = 8
B, SQ = 2, 128
BLK = 64
HQ_LOC, DH = 4, 64
DMODEL = 512
DCHUNK = HQ_LOC * DH
ROWS = (B * SQ) // N_DEV
NTOK = B * SQ


def kernel(x, Wq, K_ext, V_ext, Wo):
    my_out = lax.axis_index("i")
    Wq_my = lax.dynamic_slice(Wq, (0, my_out * DCHUNK), (DMODEL, DCHUNK))
    Wo_my = lax.dynamic_slice(Wo, (my_out * DCHUNK, 0), (DCHUNK, DMODEL))

    def body(x_ref, wq_ref, k_ref, v_ref, wo_ref, out_ref,
             partial_buf, rs_recv,
             rs_send_sems, rs_recv_sems, ag_send_sems, ag_recv_sems):
        my = lax.axis_index("i")

        barrier_sem = pltpu.get_barrier_semaphore()
        for d in range(1, N_DEV):
            tgt = lax.rem(my + d, N_DEV)
            pl.semaphore_signal(
                barrier_sem, inc=1,
                device_id=(tgt,), device_id_type=pl.DeviceIdType.MESH,
            )
        pl.semaphore_wait(barrier_sem, N_DEV - 1)

        x2d = x_ref[...].reshape(NTOK, DMODEL)
        q = jnp.dot(x2d, wq_ref[...], preferred_element_type=jnp.float32)

        blk_mask = (
            lax.broadcasted_iota(jnp.int32, (NTOK, NTOK), 0) // BLK
            == lax.broadcasted_iota(jnp.int32, (NTOK, NTOK), 1) // BLK
        )
        k_all = k_ref[...].reshape(NTOK, HQ_LOC, DH)
        v_all = v_ref[...].reshape(NTOK, HQ_LOC, DH)
        ctx_cols = []
        for h in range(HQ_LOC):
            qh = q[:, h * DH:(h + 1) * DH]
            kh = k_all[:, h, :]
            vh = v_all[:, h, :]
            s = lax.dot_general(
                qh, kh, (((1,), (1,)), ((), ())),
                preferred_element_type=jnp.float32,
            ) * 0.125
            s = jnp.where(blk_mask, s, -1e9)
            s = s - jnp.max(s, axis=-1, keepdims=True)
            w = jnp.exp(s)
            w = w / jnp.sum(w, axis=-1, keepdims=True)
            ctx_cols.append(jnp.dot(w, vh, preferred_element_type=jnp.float32))
        ctx_local = jnp.concatenate(ctx_cols, axis=1)

        partial_buf[...] = jnp.dot(
            ctx_local, wo_ref[...], preferred_element_type=jnp.float32
        )

        rs_sends = []
        for d in range(1, N_DEV):
            tgt = lax.rem(my + d, N_DEV)
            rdma = pltpu.make_async_remote_copy(
                src_ref=partial_buf.at[pl.ds(tgt * ROWS, ROWS), :],
                dst_ref=rs_recv.at[d],
                send_sem=rs_send_sems.at[d - 1],
                recv_sem=rs_recv_sems.at[d - 1],
                device_id=(tgt,),
                device_id_type=pl.DeviceIdType.MESH,
            )
            rdma.start()
            rs_sends.append(rdma)

        acc = partial_buf[pl.ds(my * ROWS, ROWS), :]
        for d in range(1, N_DEV):
            recv = pltpu.make_async_remote_copy(
                src_ref=rs_recv.at[d],
                dst_ref=rs_recv.at[d],
                send_sem=rs_send_sems.at[d - 1],
                recv_sem=rs_recv_sems.at[d - 1],
                device_id=(my,),
                device_id_type=pl.DeviceIdType.MESH,
            )
            recv.wait_recv()
            acc = acc + rs_recv[d]

        my_b = lax.div(my, 4)
        my_r = lax.rem(my, 4) * ROWS
        out_ref[my_b, pl.ds(my_r, ROWS), :] = acc
        for rdma in rs_sends:
            rdma.wait_send()

        ag_sends = []
        for d in range(1, N_DEV):
            tgt = lax.rem(my + d, N_DEV)
            rdma = pltpu.make_async_remote_copy(
                src_ref=out_ref.at[my_b, pl.ds(my_r, ROWS), :],
                dst_ref=out_ref.at[my_b, pl.ds(my_r, ROWS), :],
                send_sem=ag_send_sems.at[d - 1],
                recv_sem=ag_recv_sems.at[d - 1],
                device_id=(tgt,),
                device_id_type=pl.DeviceIdType.MESH,
            )
            rdma.start()
            ag_sends.append(rdma)
        for d in range(1, N_DEV):
            snd = lax.rem(my - d + N_DEV, N_DEV)
            recv = pltpu.make_async_remote_copy(
                src_ref=out_ref.at[lax.div(snd, 4), pl.ds(lax.rem(snd, 4) * ROWS, ROWS), :],
                dst_ref=out_ref.at[lax.div(snd, 4), pl.ds(lax.rem(snd, 4) * ROWS, ROWS), :],
                send_sem=ag_send_sems.at[d - 1],
                recv_sem=ag_recv_sems.at[d - 1],
                device_id=(snd,),
                device_id_type=pl.DeviceIdType.MESH,
            )
            recv.wait_recv()
        for rdma in ag_sends:
            rdma.wait_send()

    return pl.pallas_call(
        body,
        out_shape=jax.ShapeDtypeStruct((B, SQ, DMODEL), jnp.float32),
        in_specs=[pl.BlockSpec(memory_space=pltpu.VMEM)] * 5,
        out_specs=pl.BlockSpec(memory_space=pltpu.VMEM),
        scratch_shapes=[
            pltpu.VMEM((NTOK, DMODEL), jnp.float32),
            pltpu.VMEM((N_DEV, ROWS, DMODEL), jnp.float32),
            pltpu.SemaphoreType.DMA((N_DEV - 1,)),
            pltpu.SemaphoreType.DMA((N_DEV - 1,)),
            pltpu.SemaphoreType.DMA((N_DEV - 1,)),
            pltpu.SemaphoreType.DMA((N_DEV - 1,)),
        ],
        compiler_params=pltpu.CompilerParams(collective_id=0),
    )(x, Wq_my, K_ext, V_ext, Wo_my)
